# baseline (speedup 1.0000x reference)
import functools

import jax
import jax.numpy as jnp
from jax import lax
from jax.experimental import pallas as pl
from jax.experimental.pallas import tpu as pltpu
from jax.experimental.pallas import tpu_sc as plsc

_PEAK_T = 0.3
_OFF_T = 5.0 ** 2
_K = 64
_RPW = 4
_NS = 16
_BIG = 2 ** 30


def _rgather(a, idx):
    return lax.gather(
        a, idx[:, None], lax.GatherDimensionNumbers((), (0,), (0,)), (1,),
        mode=lax.GatherScatterMode.PROMISE_IN_BOUNDS)


def _sc_peaks_body(h_hbm, r_hbm, c0_hbm, c1_hbm, types_hbm, scores_hbm,
                   valid_hbm, buf, cmbuf, cand, cmvec, rloc, myslot, slotbuf,
                   oc0, oc1, oty, osc, ovl, slots_sh):
    cid = lax.axis_index("c")
    w = lax.axis_index("s")
    base = w * _RPW
    iota = lax.iota(jnp.int32, 16)
    fiota = iota.astype(jnp.float32)
    ninf16 = jnp.full((16,), -jnp.inf, jnp.float32)
    perms = [jnp.bitwise_xor(iota, c) for c in (1, 2, 4, 8)]

    def bmax(v):
        for pm in perms:
            v = jnp.maximum(v, _rgather(v, pm))
        return v

    def bmin(v):
        for pm in perms:
            v = jnp.minimum(v, _rgather(v, pm))
        return v

    pltpu.sync_copy(r_hbm, rloc.at[pl.ds(0, 64 * 128)])
    for i in range(_RPW + 2):
        g = jnp.clip(base + (i - 1), 0, 63)
        buf[pl.ds(i * 528, 16)] = ninf16
        buf[pl.ds(i * 528 + 512, 16)] = ninf16
        pltpu.sync_copy(h_hbm.at[pl.ds(g * 512, 512)],
                        buf.at[pl.ds(i * 528 + 8, 512)])

    def colmax_row(i, carry):
        ib = i * 528
        ob = i * 512
        for j in range(32):
            v = buf[pl.ds(ib + 8 + 16 * j, 16)]
            l = buf[pl.ds(ib + 16 * j, 16)]
            r = buf[pl.ds(ib + 16 + 16 * j, 16)]
            cmbuf[pl.ds(ob + 16 * j, 16)] = jnp.maximum(
                v, jnp.maximum(l, r))
        return carry

    lax.fori_loop(0, _RPW + 2, colmax_row, 0)

    wbase_f = (w * (_RPW * 512)).astype(jnp.float32)

    def key_row(i, carry):
        ib = i * 528
        ob = i * 512
        rb = (i - 1) * 512
        for half in range(2):
            mvec = ninf16
            for j16 in range(16):
                j = half * 16 + j16
                v = buf[pl.ds(ib + 8 + 16 * j, 16)]
                pooled = jnp.maximum(
                    cmbuf[pl.ds(ob + 16 * j, 16)],
                    jnp.maximum(cmbuf[pl.ds(ob - 512 + 16 * j, 16)],
                                cmbuf[pl.ds(ob + 512 + 16 * j, 16)]))
                msk = (v == pooled) & (v > _PEAK_T)
                gflat_f = wbase_f + (rb + 16 * j).astype(jnp.float32) + fiota
                key = jnp.where(msk, v, -1.0 - gflat_f)
                cand[pl.ds(rb + 16 * j, 16)] = key
                bc = bmax(key)
                mvec = jnp.where(iota == j16, bc, mvec)
            cmvec[pl.ds((i - 1) * 32 + half * 16, 16)] = mvec
        return carry

    lax.fori_loop(1, _RPW + 1, key_row, 0)

    tk = [jnp.zeros((16,), jnp.float32) for _ in range(4)]
    ti = [jnp.zeros((16,), jnp.float32) for _ in range(4)]

    def round_body(t, carry):
        tk = list(carry[:4])
        ti = list(carry[4:8])
        cm8 = [cmvec[pl.ds(v8 * 16, 16)] for v8 in range(8)]
        fold = cm8[0]
        for v8 in range(1, 8):
            fold = jnp.maximum(fold, cm8[v8])
        k_loc = bmax(fold)
        qc = jnp.full((16,), 999.0, jnp.float32)
        for v8 in range(8):
            qc = jnp.minimum(qc, jnp.where(cm8[v8] == k_loc,
                                           fiota + 16.0 * v8, 999.0))
        qv = bmin(qc)
        q_s = qv[0].astype(jnp.int32)
        v = cand[pl.ds(q_s * 16, 16)]
        lv = bmin(jnp.where(v == k_loc, fiota, 999.0))
        gidx_mine = wbase_f + qv * 16.0 + lv
        myslot[pl.ds(0, 16)] = k_loc
        myslot[pl.ds(16, 16)] = gidx_mine
        pltpu.sync_copy(myslot, slots_sh.at[pl.ds(w * 32, 32)])
        plsc.subcore_barrier()
        pltpu.sync_copy(slots_sh, slotbuf)
        dk = jnp.zeros((16,), jnp.float32)
        di = jnp.zeros((16,), jnp.float32)
        for j in range(_NS):
            dk = jnp.where(iota == j, slotbuf[pl.ds(j * 32, 16)], dk)
            di = jnp.where(iota == j, slotbuf[pl.ds(j * 32 + 16, 16)], di)
        gk = bmax(dk)
        gidx = bmin(jnp.where(dk == gk, di, 1e9))
        iswin = gidx == gidx_mine
        newv = jnp.where(iswin & (fiota == lv), -1.0 - gidx_mine, v)
        cand[pl.ds(q_s * 16, 16)] = newv
        nmx = bmax(newv)
        vecid = lax.shift_right_arithmetic(q_s, 4)
        cmv = cmvec[pl.ds(vecid * 16, 16)]
        cmvec[pl.ds(vecid * 16, 16)] = jnp.where(
            iswin & (iota == (q_s % 16)), nmx, cmv)
        for vec in range(4):
            sel = (iota + 16 * vec) == t
            tk[vec] = jnp.where(sel, gk, tk[vec])
            ti[vec] = jnp.where(sel, gidx, ti[vec])
        plsc.subcore_barrier()
        return tuple(tk + ti)

    carry = lax.fori_loop(0, _K, round_body, tuple(tk + ti))
    tk = carry[:4]
    ti = carry[4:8]

    for vec in range(4):
        key = tk[vec]
        gidx = ti[vec].astype(jnp.int32)
        validb = key > 0.0
        score = jnp.where(validb, key, 0.0)
        pix = lax.shift_right_arithmetic(gidx, 3)
        py = lax.shift_right_arithmetic(pix, 6).astype(jnp.float32)
        px = (pix % 64).astype(jnp.float32)
        r0 = jnp.zeros((16,), jnp.float32)
        r1 = jnp.zeros((16,), jnp.float32)
        for lane in range(16):
            pix_s = pix[lane]
            rb = rloc[pl.ds(pix_s * 2, 16)]
            r0 = jnp.where(iota == lane, rb[0], r0)
            r1 = jnp.where(iota == lane, rb[1], r1)
        oc0[pl.ds(16 * vec, 16)] = (px + r1) * 8.0
        oc1[pl.ds(16 * vec, 16)] = (py + r0) * 8.0
        oty[pl.ds(16 * vec, 16)] = gidx % 8
        osc[pl.ds(16 * vec, 16)] = score
        ovl[pl.ds(16 * vec, 16)] = jnp.where(validb, 1.0, 0.0)

    @pl.when((cid == 0) & (w == 0))
    def _():
        pltpu.sync_copy(oc0, c0_hbm)
        pltpu.sync_copy(oc1, c1_hbm)
        pltpu.sync_copy(oty, types_hbm)
        pltpu.sync_copy(osc, scores_hbm)
        pltpu.sync_copy(ovl, valid_hbm)


def _sc_peaks(h2, r2):
    f32 = jnp.float32
    i32 = jnp.int32
    mesh = plsc.VectorSubcoreMesh(core_axis_name="c", subcore_axis_name="s")
    fn = functools.partial(
        pl.kernel,
        mesh=mesh,
        out_type=[
            jax.ShapeDtypeStruct((_K,), f32),
            jax.ShapeDtypeStruct((_K,), f32),
            jax.ShapeDtypeStruct((_K,), i32),
            jax.ShapeDtypeStruct((_K,), f32),
            jax.ShapeDtypeStruct((_K,), f32),
        ],
        scratch_types=[
            pltpu.VMEM(((_RPW + 2) * 528,), f32),
            pltpu.VMEM(((_RPW + 2) * 512,), f32),
            pltpu.VMEM((_RPW * 512,), f32),
            pltpu.VMEM((128,), f32),
            pltpu.VMEM((64 * 128 + 16,), f32),
            pltpu.VMEM((32,), f32),
            pltpu.VMEM((_NS * 32,), f32),
            pltpu.VMEM((_K,), f32),
            pltpu.VMEM((_K,), f32),
            pltpu.VMEM((_K,), i32),
            pltpu.VMEM((_K,), f32),
            pltpu.VMEM((_K,), f32),
            pltpu.VMEM_SHARED((_NS * 32,), f32),
        ],
    )(_sc_peaks_body)
    return fn(h2, r2)


def _dense_kernel(st_ref, iy_ref, ix_ref, c0_ref, c1_ref, valid_ref,
                  aff_ref, osc_ref, coy_ref, cox_ref):
    rows = iy_ref.shape[0]
    i = pl.program_id(0)
    s0 = st_ref[0]
    m = st_ref[1]
    for c in range(2, 8):
        m = jnp.maximum(m, st_ref[c])
    non_bg = m > s0
    yy = (lax.broadcasted_iota(jnp.int32, (rows, 512), 0)
          + i * rows).astype(jnp.float32)
    xx = lax.broadcasted_iota(jnp.int32, (rows, 512), 1).astype(jnp.float32)
    o0 = iy_ref[...] + yy
    o1 = ix_ref[...] + xx
    mind = jnp.full((rows, 512), jnp.inf, jnp.float32)
    amin = jnp.zeros((rows, 512), jnp.int32)
    for k in range(_K):
        d0 = o0 - c0_ref[0, k]
        d1 = o1 - c1_ref[0, k]
        d = d0 * d0 + d1 * d1
        d = jnp.where(valid_ref[0, k] > 0.5, d, 1e30)
        upd = d < mind
        amin = jnp.where(upd, k, amin)
        mind = jnp.where(upd, d, mind)
    validp = non_bg & (mind < _OFF_T)
    aff_ref[...] = jnp.where(validp, amin, -1)
    osc_ref[...] = jnp.where(validp, mind, 0.0)
    coy_ref[...] = jnp.where(validp, yy, 0.0)
    cox_ref[...] = jnp.where(validp, xx, 0.0)


@jax.jit
def kernel(hmap, rreg, iseg, sseg):
    h2 = hmap[0].reshape(64 * 512)
    r2 = rreg[0].reshape(64 * 128)
    f32 = jnp.float32
    c0, c1, types, scores, valid = _sc_peaks(h2, r2)
    c0r = c0.reshape(1, _K)
    c1r = c1.reshape(1, _K)
    validr = valid.reshape(1, _K)

    sseg_t = jnp.transpose(sseg[0], (2, 0, 1))
    iy = iseg[0, :, :, 1]
    ix = iseg[0, :, :, 0]
    R = 64
    G = 512 // R
    smem = pl.BlockSpec(memory_space=pltpu.SMEM)
    aff, osc, coy, cox = pl.pallas_call(
        _dense_kernel,
        grid=(G,),
        in_specs=[
            pl.BlockSpec((8, R, 512), lambda i: (0, i, 0)),
            pl.BlockSpec((R, 512), lambda i: (i, 0)),
            pl.BlockSpec((R, 512), lambda i: (i, 0)),
            smem, smem, smem,
        ],
        out_specs=[
            pl.BlockSpec((R, 512), lambda i: (i, 0)),
            pl.BlockSpec((R, 512), lambda i: (i, 0)),
            pl.BlockSpec((R, 512), lambda i: (i, 0)),
            pl.BlockSpec((R, 512), lambda i: (i, 0)),
        ],
        out_shape=[
            jax.ShapeDtypeStruct((512, 512), jnp.int32),
            jax.ShapeDtypeStruct((512, 512), f32),
            jax.ShapeDtypeStruct((512, 512), f32),
            jax.ShapeDtypeStruct((512, 512), f32),
        ],
    )(sseg_t, iy, ix, c0r, c1r, validr)

    centroids = jnp.concatenate([c0.reshape(_K, 1), c1.reshape(_K, 1)],
                                axis=1)
    coords = jnp.stack([coy, cox], axis=-1).reshape(-1, 2)
    return (coords, aff.reshape(-1), centroids, types,
            scores, osc.reshape(-1))
